# baseline (speedup 1.0000x reference)
import jax
import jax.numpy as jnp
from jax import lax
from jax.experimental import pallas as pl
from jax.experimental.pallas import tpu as pltpu

ROWS = 16384
COLS = 201
BLK = 512

_DOT_T = (((1,), (1,)), ((), ()))


def _body(inv_t_ref, y_ref, o_ref):
    inv_t = inv_t_ref[0]
    y = y_ref[...] * inv_t
    e = jnp.exp(y)
    col = lax.broadcasted_iota(jnp.int32, (BLK, COLS), 1)
    keep = (col == 0) | (y > 0.0)
    c = jnp.where(keep, e, 0.0)
    ones = jnp.ones((1, COLS), jnp.float32)
    e1 = (lax.broadcasted_iota(jnp.int32, (1, COLS), 1) == 0).astype(jnp.float32)
    s = lax.dot_general(ones, c, _DOT_T, preferred_element_type=jnp.float32)
    y0 = lax.dot_general(e1, y, _DOT_T, preferred_element_type=jnp.float32)
    o_ref[...] = (jnp.log(s) - y0)[0]


def kernel(y_pred, mask_zeros, temperature_):
    del mask_zeros
    inv_t = (1.0 / temperature_).astype(jnp.float32)
    grid = (ROWS // BLK,)
    out = pl.pallas_call(
        _body,
        grid=grid,
        in_specs=[
            pl.BlockSpec(memory_space=pltpu.SMEM),
            pl.BlockSpec((BLK, COLS), lambda i: (i, 0)),
        ],
        out_specs=pl.BlockSpec((BLK,), lambda i: (i,)),
        out_shape=jax.ShapeDtypeStruct((ROWS,), jnp.float32),
    )(inv_t, y_pred)
    return (out, 0.0)


# speedup vs baseline: 1.5108x; 1.5108x over previous
import jax
import jax.numpy as jnp
from jax import lax
from jax.experimental import pallas as pl
from jax.experimental.pallas import tpu as pltpu

ROWS = 16384
COLS = 201
BLK = 8192

_DOT_T = (((1,), (1,)), ((), ()))


def _body(inv_t_ref, y_ref, o_ref):
    inv_t = inv_t_ref[0]
    y = y_ref[...] * inv_t
    e = jnp.exp(y)
    col = lax.broadcasted_iota(jnp.int32, (BLK, COLS), 1)
    keep = (col == 0) | (y > 0.0)
    c = jnp.where(keep, e, 0.0)
    ones = jnp.ones((1, COLS), jnp.float32)
    e1 = (lax.broadcasted_iota(jnp.int32, (1, COLS), 1) == 0).astype(jnp.float32)
    s = lax.dot_general(ones, c, _DOT_T, preferred_element_type=jnp.float32)
    y0 = lax.dot_general(e1, y, _DOT_T, preferred_element_type=jnp.float32)
    o_ref[...] = (jnp.log(s) - y0)[0]


def kernel(y_pred, mask_zeros, temperature_):
    del mask_zeros
    inv_t = (1.0 / temperature_).astype(jnp.float32)
    grid = (ROWS // BLK,)
    out = pl.pallas_call(
        _body,
        grid=grid,
        in_specs=[
            pl.BlockSpec(memory_space=pltpu.SMEM),
            pl.BlockSpec((BLK, COLS), lambda i: (i, 0)),
        ],
        out_specs=pl.BlockSpec((BLK,), lambda i: (i,)),
        out_shape=jax.ShapeDtypeStruct((ROWS,), jnp.float32),
    )(inv_t, y_pred)
    return (out, 0.0)
